# Initial kernel scaffold; baseline (speedup 1.0000x reference)
import jax
import jax.numpy as jnp
from jax.experimental import pallas as pl


def kernel(x, edge_index, batch, W1_f, b1_f, W2_f, b2_f, W1_r, b1_r, W2_r, b2_r, Wb, bb, Wm, bm, Wmean, bmean):
    raise NotImplementedError("write your pallas kernel here")


# speedup vs baseline: 8.4033x; 8.4033x over previous
import functools

import jax
import jax.numpy as jnp
from jax import lax
from jax.experimental import pallas as pl
from jax.experimental.pallas import tpu as pltpu
from jax.experimental.pallas import tpu_sc as plsc

_N = 10000
_E = 320000
_D = 128
_H = 32
_G = 64

_NC = 2
_NS = 16
_EW = _E // (_NC * _NS)
_K = 80
_NCHUNK = _EW // _K
_NP = 10240
_RS = _NP // _NS

_BN = 1000
_NB = _N // _BN


def _proj_body(x_ref, w_ref, z_ref):
    z_ref[...] = jnp.dot(x_ref[...], w_ref[...],
                         preferred_element_type=jnp.float32)


def _project(x, w1cat):
    return pl.pallas_call(
        _proj_body,
        grid=(_NB,),
        in_specs=[
            pl.BlockSpec((_BN, _D), lambda i: (i, 0)),
            pl.BlockSpec((_D, 2 * _H), lambda i: (0, 0)),
        ],
        out_specs=pl.BlockSpec((_BN, 2 * _H), lambda i: (i, 0)),
        out_shape=jax.ShapeDtypeStruct((_N, 2 * _H), jnp.float32),
    )(x, w1cat)


def _agg_body(zf_h, zr_h, src_h, dst_h, outf_h, outr_h,
              src_v, dst_v, rows_f, rows_r, zbuf,
              accf, accr, sem_f, sem_r):
    c = lax.axis_index("c")
    s = lax.axis_index("s")

    def _zrow(i, carry):
        zbuf[i, pl.ds(0, 16)] = jnp.zeros((16,), jnp.float32)
        zbuf[i, pl.ds(16, 16)] = jnp.zeros((16,), jnp.float32)
        return carry
    lax.fori_loop(0, _RS, _zrow, 0)
    row0 = s * _RS
    pltpu.sync_copy(zbuf, accf.at[pl.ds(row0, _RS)])
    pltpu.sync_copy(zbuf, accr.at[pl.ds(row0, _RS)])
    plsc.subcore_barrier()

    base = (c * _NS + s) * _EW

    def _chunk(k, carry):
        off = base + k * _K
        pltpu.sync_copy(src_h.at[pl.ds(off, _K)], src_v)
        pltpu.sync_copy(dst_h.at[pl.ds(off, _K)], dst_v)
        cpf = pltpu.async_copy(zf_h.at[src_v], rows_f, sem_f)
        cpr = pltpu.async_copy(zr_h.at[dst_v], rows_r, sem_r)
        cpf.wait()
        pltpu.sync_copy(rows_f, accf.at[dst_v], add=True)
        cpr.wait()
        pltpu.sync_copy(rows_r, accr.at[src_v], add=True)
        return carry
    lax.fori_loop(0, _NCHUNK, _chunk, 0)

    plsc.subcore_barrier()
    pltpu.sync_copy(accf.at[pl.ds(row0, _RS)], outf_h.at[c, pl.ds(row0, _RS)])
    pltpu.sync_copy(accr.at[pl.ds(row0, _RS)], outr_h.at[c, pl.ds(row0, _RS)])


def _aggregate(z_f, z_r, src, dst):
    mesh = plsc.VectorSubcoreMesh(core_axis_name="c", subcore_axis_name="s")
    kern = functools.partial(
        pl.kernel,
        out_type=(
            jax.ShapeDtypeStruct((_NC, _NP, _H), jnp.float32),
            jax.ShapeDtypeStruct((_NC, _NP, _H), jnp.float32),
        ),
        mesh=mesh,
        compiler_params=pltpu.CompilerParams(use_tc_tiling_on_sc=False),
        scratch_types=[
            pltpu.VMEM((_K,), jnp.int32),
            pltpu.VMEM((_K,), jnp.int32),
            pltpu.VMEM((_K, _H), jnp.float32),
            pltpu.VMEM((_K, _H), jnp.float32),
            pltpu.VMEM((_RS, _H), jnp.float32),
            pltpu.VMEM_SHARED((_NP, _H), jnp.float32),
            pltpu.VMEM_SHARED((_NP, _H), jnp.float32),
            pltpu.SemaphoreType.DMA,
            pltpu.SemaphoreType.DMA,
        ],
    )(_agg_body)
    return kern(z_f, z_r, src, dst)


def _head_body(z_ref, af0_ref, af1_ref, ar0_ref, ar1_ref, batch_ref,
               b1_ref, w2f_ref, b2f_ref, w2r_ref, b2r_ref,
               wb_ref, bb_ref, wm_ref, bm_ref, wmean_ref, bmean_ref,
               out_ref, s_acc, c_acc):
    i = pl.program_id(0)

    @pl.when(i == 0)
    def _init():
        s_acc[...] = jnp.zeros_like(s_acc)
        c_acc[...] = jnp.zeros_like(c_acc)

    agg = jnp.concatenate(
        [af0_ref[0] + af1_ref[0], ar0_ref[0] + ar1_ref[0]], axis=1)
    r = jnp.maximum(z_ref[...] + agg + b1_ref[...], 0.0)

    gids = lax.broadcasted_iota(jnp.int32, (1, _G), 1)
    oh = (batch_ref[...] == gids).astype(jnp.float32)
    dn = (((0,), (0,)), ((), ()))
    s_acc[...] += lax.dot_general(oh, r, dn,
                                  preferred_element_type=jnp.float32)
    ones = jnp.ones((_BN, 1), jnp.float32)
    c_acc[...] += lax.dot_general(oh, ones, dn,
                                  preferred_element_type=jnp.float32)

    @pl.when(i == _NB - 1)
    def _finish():
        mean = s_acc[...] / jnp.maximum(c_acc[...], 1.0)
        pf = jnp.dot(mean[:, :_H], w2f_ref[...],
                     preferred_element_type=jnp.float32) + b2f_ref[...]
        pr = jnp.dot(mean[:, _H:], w2r_ref[...],
                     preferred_element_type=jnp.float32) + b2r_ref[...]
        emb = jnp.concatenate([pf, pr], axis=1)
        h = jnp.maximum(jnp.dot(emb, wb_ref[...],
                                preferred_element_type=jnp.float32)
                        + bb_ref[...], 0.0)
        h = jnp.maximum(jnp.dot(h, wm_ref[...],
                                preferred_element_type=jnp.float32)
                        + bm_ref[...], 0.0)
        m = jnp.dot(h, wmean_ref[...],
                    preferred_element_type=jnp.float32) + bmean_ref[...]
        out_ref[...] = jax.nn.sigmoid(m)


def _pool_head(z, aggf, aggr, batch2d, b1cat, W2_f, b2_f, W2_r, b2_r,
               Wb, bb, Wm, bm, Wmean, bmean):
    full = lambda shape: pl.BlockSpec(shape, lambda i: tuple(0 for _ in shape))
    return pl.pallas_call(
        _head_body,
        grid=(_NB,),
        in_specs=[
            pl.BlockSpec((_BN, 2 * _H), lambda i: (i, 0)),
            pl.BlockSpec((1, _BN, _H), lambda i: (0, i, 0)),
            pl.BlockSpec((1, _BN, _H), lambda i: (1, i, 0)),
            pl.BlockSpec((1, _BN, _H), lambda i: (0, i, 0)),
            pl.BlockSpec((1, _BN, _H), lambda i: (1, i, 0)),
            pl.BlockSpec((_BN, 1), lambda i: (i, 0)),
            full((1, 2 * _H)),
            full((_H, _H)), full((1, _H)),
            full((_H, _H)), full((1, _H)),
            full((2 * _H, _H)), full((1, _H)),
            full((_H, 16)), full((1, 16)),
            full((16, 1)), full((1, 1)),
        ],
        out_specs=pl.BlockSpec((_G, 1), lambda i: (0, 0)),
        out_shape=jax.ShapeDtypeStruct((_G, 1), jnp.float32),
        scratch_shapes=[
            pltpu.VMEM((_G, 2 * _H), jnp.float32),
            pltpu.VMEM((_G, 1), jnp.float32),
        ],
    )(z, aggf, aggf, aggr, aggr, batch2d, b1cat, W2_f, b2_f, W2_r, b2_r,
      Wb, bb, Wm, bm, Wmean, bmean)


def kernel(x, edge_index, batch, W1_f, b1_f, W2_f, b2_f, W1_r, b1_r,
           W2_r, b2_r, Wb, bb, Wm, bm, Wmean, bmean):
    w1cat = jnp.concatenate([W1_f, W1_r], axis=1)
    z = _project(x, w1cat)
    z_f = z[:, :_H]
    z_r = z[:, _H:]
    src = edge_index[0]
    dst = edge_index[1]
    aggf, aggr = _aggregate(z_f, z_r, src, dst)

    batch2d = batch.reshape(_N, 1)
    b1cat = jnp.concatenate([b1_f, b1_r]).reshape(1, 2 * _H)
    return _pool_head(
        z, aggf, aggr, batch2d,
        b1cat, W2_f, b2_f.reshape(1, _H), W2_r, b2_r.reshape(1, _H),
        Wb, bb.reshape(1, _H), Wm, bm.reshape(1, 16),
        Wmean, bmean.reshape(1, 1))


# speedup vs baseline: 20.0981x; 2.3917x over previous
import functools

import jax
import jax.numpy as jnp
from jax import lax
from jax.experimental import pallas as pl
from jax.experimental.pallas import tpu as pltpu
from jax.experimental.pallas import tpu_sc as plsc

_N = 10000
_E = 320000
_D = 128
_H = 32
_G = 64

_NC = 2
_NS = 16
_EW = _E // (_NC * _NS)
_K = 80
_NCHUNK = _EW // _K
_NP = 10240
_RS = _NP // _NS

_BN = 1000
_NB = _N // _BN


def _proj_body(x_ref, w_ref, z_ref):
    z_ref[...] = jnp.dot(x_ref[...], w_ref[...],
                         preferred_element_type=jnp.float32)


def _project(x, w1cat):
    return pl.pallas_call(
        _proj_body,
        grid=(_NB,),
        in_specs=[
            pl.BlockSpec((_BN, _D), lambda i: (i, 0)),
            pl.BlockSpec((_D, 2 * _H), lambda i: (0, 0)),
        ],
        out_specs=pl.BlockSpec((_BN, 2 * _H), lambda i: (i, 0)),
        out_shape=jax.ShapeDtypeStruct((_N, 2 * _H), jnp.float32),
    )(x, w1cat)


_NBUF = 5


def _agg_body(zf_h, zr_h, src_h, dst_h, outf_h, outr_h, *refs):
    (src_v, dst_v, zbuf, accf, accr) = refs[:5]
    rows_f = refs[5:5 + _NBUF]
    rows_r = refs[5 + _NBUF:5 + 2 * _NBUF]
    sgf = refs[5 + 2 * _NBUF:5 + 3 * _NBUF]
    sgr = refs[5 + 3 * _NBUF:5 + 4 * _NBUF]
    ssf = refs[5 + 4 * _NBUF:5 + 5 * _NBUF]
    ssr = refs[5 + 5 * _NBUF:5 + 6 * _NBUF]

    c = lax.axis_index("c")
    s = lax.axis_index("s")
    wid = c * _NS + s

    pltpu.sync_copy(src_h.at[wid], src_v)
    pltpu.sync_copy(dst_h.at[wid], dst_v)

    def _zrow(i, carry):
        zbuf[i, pl.ds(0, 16)] = jnp.zeros((16,), jnp.float32)
        zbuf[i, pl.ds(16, 16)] = jnp.zeros((16,), jnp.float32)
        return carry
    lax.fori_loop(0, _RS, _zrow, 0)
    row0 = s * _RS
    pltpu.sync_copy(zbuf, accf.at[pl.ds(row0, _RS)])
    pltpu.sync_copy(zbuf, accr.at[pl.ds(row0, _RS)])
    plsc.subcore_barrier()

    def _gather(k, b):
        cf = pltpu.async_copy(zf_h.at[src_v.at[k]], rows_f[b], sgf[b])
        cr = pltpu.async_copy(zr_h.at[dst_v.at[k]], rows_r[b], sgr[b])
        return cf, cr

    def _wait_gather(k, b):
        pltpu.make_async_copy(zf_h.at[src_v.at[k]], rows_f[b], sgf[b]).wait()
        pltpu.make_async_copy(zr_h.at[dst_v.at[k]], rows_r[b], sgr[b]).wait()

    def _scatter(k, b):
        cf = pltpu.async_copy(rows_f[b], accf.at[dst_v.at[k]], ssf[b],
                              add=True)
        cr = pltpu.async_copy(rows_r[b], accr.at[src_v.at[k]], ssr[b],
                              add=True)
        return cf, cr

    for b in range(_NBUF):
        _gather(b, b)

    def _steady(i, carry):
        for b in range(_NBUF):
            k = i * _NBUF + b
            _wait_gather(k, b)
            cf, cr = _scatter(k, b)
            cf.wait()
            cr.wait()
            _gather(k + _NBUF, b)
        return carry
    lax.fori_loop(0, _NCHUNK // _NBUF - 1, _steady, 0)

    for b in range(_NBUF):
        k = _NCHUNK - _NBUF + b
        _wait_gather(k, b)
        cf, cr = _scatter(k, b)
        cf.wait()
        cr.wait()

    plsc.subcore_barrier()
    pltpu.sync_copy(accf.at[pl.ds(row0, _RS)], outf_h.at[c, pl.ds(row0, _RS)])
    pltpu.sync_copy(accr.at[pl.ds(row0, _RS)], outr_h.at[c, pl.ds(row0, _RS)])


def _aggregate(z_f, z_r, src3, dst3):
    mesh = plsc.VectorSubcoreMesh(core_axis_name="c", subcore_axis_name="s")
    kern = functools.partial(
        pl.kernel,
        out_type=(
            jax.ShapeDtypeStruct((_NC, _NP, _H), jnp.float32),
            jax.ShapeDtypeStruct((_NC, _NP, _H), jnp.float32),
        ),
        mesh=mesh,
        compiler_params=pltpu.CompilerParams(use_tc_tiling_on_sc=False),
        scratch_types=(
            [pltpu.VMEM((_NCHUNK, _K), jnp.int32),
             pltpu.VMEM((_NCHUNK, _K), jnp.int32),
             pltpu.VMEM((_RS, _H), jnp.float32),
             pltpu.VMEM_SHARED((_NP, _H), jnp.float32),
             pltpu.VMEM_SHARED((_NP, _H), jnp.float32)]
            + [pltpu.VMEM((_K, _H), jnp.float32)] * (2 * _NBUF)
            + [pltpu.SemaphoreType.DMA] * (4 * _NBUF)
        ),
    )(_agg_body)
    return kern(z_f, z_r, src3, dst3)


def _head_body(z_ref, af0_ref, af1_ref, ar0_ref, ar1_ref, batch_ref,
               b1_ref, w2f_ref, b2f_ref, w2r_ref, b2r_ref,
               wb_ref, bb_ref, wm_ref, bm_ref, wmean_ref, bmean_ref,
               out_ref, s_acc, c_acc):
    i = pl.program_id(0)

    @pl.when(i == 0)
    def _init():
        s_acc[...] = jnp.zeros_like(s_acc)
        c_acc[...] = jnp.zeros_like(c_acc)

    agg = jnp.concatenate(
        [af0_ref[0] + af1_ref[0], ar0_ref[0] + ar1_ref[0]], axis=1)
    r = jnp.maximum(z_ref[...] + agg + b1_ref[...], 0.0)

    gids = lax.broadcasted_iota(jnp.int32, (1, _G), 1)
    oh = (batch_ref[...] == gids).astype(jnp.float32)
    dn = (((0,), (0,)), ((), ()))
    s_acc[...] += lax.dot_general(oh, r, dn,
                                  preferred_element_type=jnp.float32)
    ones = jnp.ones((_BN, 1), jnp.float32)
    c_acc[...] += lax.dot_general(oh, ones, dn,
                                  preferred_element_type=jnp.float32)

    @pl.when(i == _NB - 1)
    def _finish():
        mean = s_acc[...] / jnp.maximum(c_acc[...], 1.0)
        pf = jnp.dot(mean[:, :_H], w2f_ref[...],
                     preferred_element_type=jnp.float32) + b2f_ref[...]
        pr = jnp.dot(mean[:, _H:], w2r_ref[...],
                     preferred_element_type=jnp.float32) + b2r_ref[...]
        emb = jnp.concatenate([pf, pr], axis=1)
        h = jnp.maximum(jnp.dot(emb, wb_ref[...],
                                preferred_element_type=jnp.float32)
                        + bb_ref[...], 0.0)
        h = jnp.maximum(jnp.dot(h, wm_ref[...],
                                preferred_element_type=jnp.float32)
                        + bm_ref[...], 0.0)
        m = jnp.dot(h, wmean_ref[...],
                    preferred_element_type=jnp.float32) + bmean_ref[...]
        out_ref[...] = jax.nn.sigmoid(m)


def _pool_head(z, aggf, aggr, batch2d, b1cat, W2_f, b2_f, W2_r, b2_r,
               Wb, bb, Wm, bm, Wmean, bmean):
    full = lambda shape: pl.BlockSpec(shape, lambda i: tuple(0 for _ in shape))
    return pl.pallas_call(
        _head_body,
        grid=(_NB,),
        in_specs=[
            pl.BlockSpec((_BN, 2 * _H), lambda i: (i, 0)),
            pl.BlockSpec((1, _BN, _H), lambda i: (0, i, 0)),
            pl.BlockSpec((1, _BN, _H), lambda i: (1, i, 0)),
            pl.BlockSpec((1, _BN, _H), lambda i: (0, i, 0)),
            pl.BlockSpec((1, _BN, _H), lambda i: (1, i, 0)),
            pl.BlockSpec((_BN, 1), lambda i: (i, 0)),
            full((1, 2 * _H)),
            full((_H, _H)), full((1, _H)),
            full((_H, _H)), full((1, _H)),
            full((2 * _H, _H)), full((1, _H)),
            full((_H, 16)), full((1, 16)),
            full((16, 1)), full((1, 1)),
        ],
        out_specs=pl.BlockSpec((_G, 1), lambda i: (0, 0)),
        out_shape=jax.ShapeDtypeStruct((_G, 1), jnp.float32),
        scratch_shapes=[
            pltpu.VMEM((_G, 2 * _H), jnp.float32),
            pltpu.VMEM((_G, 1), jnp.float32),
        ],
    )(z, aggf, aggf, aggr, aggr, batch2d, b1cat, W2_f, b2_f, W2_r, b2_r,
      Wb, bb, Wm, bm, Wmean, bmean)


def kernel(x, edge_index, batch, W1_f, b1_f, W2_f, b2_f, W1_r, b1_r,
           W2_r, b2_r, Wb, bb, Wm, bm, Wmean, bmean):
    w1cat = jnp.concatenate([W1_f, W1_r], axis=1)
    z = _project(x, w1cat)
    z_f = z[:, :_H]
    z_r = z[:, _H:]
    src3 = edge_index[0].reshape(_NC * _NS, _NCHUNK, _K)
    dst3 = edge_index[1].reshape(_NC * _NS, _NCHUNK, _K)
    aggf, aggr = _aggregate(z_f, z_r, src3, dst3)

    batch2d = batch.reshape(_N, 1)
    b1cat = jnp.concatenate([b1_f, b1_r]).reshape(1, 2 * _H)
    return _pool_head(
        z, aggf, aggr, batch2d,
        b1cat, W2_f, b2_f.reshape(1, _H), W2_r, b2_r.reshape(1, _H),
        Wb, bb.reshape(1, _H), Wm, bm.reshape(1, 16),
        Wmean, bmean.reshape(1, 1))
